# baseline (speedup 1.0000x reference)
import functools

import jax
import jax.numpy as jnp
from jax import lax
from jax.experimental import pallas as pl
from jax.experimental.pallas import tpu as pltpu
from jax.experimental.pallas import tpu_sc as plsc

N = 4096
E = 262144
NC, NS, L = 2, 16, 16
NW = NC * NS
EPW = E // NW
GW = 1024
NCH = EPW // GW
VPC = GW // L
ECH = EPW // 128


def _sc_segment_partials(xflat, adj, edges3):
  mesh = plsc.VectorSubcoreMesh(
      core_axis_name="c", subcore_axis_name="s", num_cores=NC,
      num_subcores=NS)

  @functools.partial(
      pl.kernel,
      out_type=jax.ShapeDtypeStruct((NW, N), jnp.float32),
      mesh=mesh,
      compiler_params=pltpu.CompilerParams(needs_layout_passes=False),
      scratch_types=[
          pltpu.VMEM((ECH, 2, 128), jnp.int32),
          pltpu.VMEM((EPW,), jnp.float32),
          pltpu.VMEM((EPW,), jnp.int32),
          pltpu.VMEM((EPW,), jnp.float32),
          pltpu.VMEM((N,), jnp.float32),
          pltpu.SemaphoreType.DMA,
          pltpu.SemaphoreType.DMA,
          pltpu.SemaphoreType.DMA,
      ],
  )
  def body(xflat_h, adj_h, edges_h, out_h,
           edges_v, adj_v, idx_v, gath_v, acc_v, sem_in, sem_a, sem_b):
    wid = lax.axis_index("s") * NC + lax.axis_index("c")
    cp_e = pltpu.async_copy(edges_h.at[pl.ds(wid * ECH, ECH)], edges_v,
                            sem_in)
    cp_a = pltpu.async_copy(adj_h.at[pl.ds(wid * EPW, EPW)], adj_v, sem_in)
    cp_e.wait()
    cp_a.wait()

    def idx_row(t, carry):
      for k in range(8):
        r = edges_v[t, 0, pl.ds(k * L, L)]
        c = edges_v[t, 1, pl.ds(k * L, L)]
        idx_v[pl.ds(t * 128 + k * L, L)] = (
            ((r + ((r >> 3) * 248)) << 7) + (c + ((c >> 7) * 896)))
      return carry

    RPC = GW // 128
    gathers = []
    for j in range(NCH):
      lax.fori_loop(j * RPC, (j + 1) * RPC, idx_row, 0)
      s = pl.ds(j * GW, GW)
      gathers.append(pltpu.async_copy(
          xflat_h.at[idx_v.at[s]], gath_v.at[s],
          sem_a if j < NCH // 2 else sem_b))

    def zero(i, carry):
      for k in range(4):
        acc_v[pl.ds(i * 4 * L + k * L, L)] = jnp.zeros((L,), jnp.float32)
      return carry
    lax.fori_loop(0, N // (4 * L), zero, 0)

    def scat_row(t, carry):
      for k in range(8):
        s = pl.ds(t * 128 + k * L, L)
        r = edges_v[t, 0, pl.ds(k * L, L)]
        plsc.addupdate_scatter(acc_v, [r], gath_v[s] * adj_v[s])
      return carry

    for cp in gathers[:NCH // 2]:
      cp.wait()
    lax.fori_loop(0, (NCH // 2) * RPC, scat_row, 0)
    for cp in gathers[NCH // 2:]:
      cp.wait()
    lax.fori_loop((NCH // 2) * RPC, NCH * RPC, scat_row, 0)

    pltpu.sync_copy(acc_v, out_h.at[wid])

  return body(xflat, adj, edges3)


def _tc_combine_softmax(partials):
  def body(p_ref, araw_ref, alpha_ref):
    a = jnp.sum(p_ref[...], axis=0, keepdims=True)
    araw_ref[...] = a
    m = jnp.max(a)
    e = jnp.exp(a - m)
    alpha_ref[...] = e / jnp.sum(e)

  araw, alpha = pl.pallas_call(
      body,
      out_shape=(jax.ShapeDtypeStruct((1, N), jnp.float32),
                 jax.ShapeDtypeStruct((1, N), jnp.float32)),
  )(partials)
  return araw, alpha


def kernel(data_input, adj_values, edge_index):
  xflat = (data_input.reshape(N // 8, 8, N // 128, 128)
           .swapaxes(1, 2).reshape(N * N))
  edges3 = (edge_index.reshape(2, E // 128, 128).swapaxes(0, 1))
  partials = _sc_segment_partials(xflat, adj_values, edges3)
  araw, alpha = _tc_combine_softmax(partials)
  return alpha.reshape(N), araw.reshape(N)


# speedup vs baseline: 1.0862x; 1.0862x over previous
import functools

import jax
import jax.numpy as jnp
from jax import lax
from jax.experimental import pallas as pl
from jax.experimental.pallas import tpu as pltpu
from jax.experimental.pallas import tpu_sc as plsc

N = 4096
E = 262144
NC, NS, L = 2, 16, 16
NW = NC * NS
EPW = E // NW
GW = 1024
NCH = EPW // GW
VPC = GW // L
ECH = EPW // 128


def _sc_segment_partials(xflat, adj, edges3):
  mesh = plsc.VectorSubcoreMesh(
      core_axis_name="c", subcore_axis_name="s", num_cores=NC,
      num_subcores=NS)

  @functools.partial(
      pl.kernel,
      out_type=jax.ShapeDtypeStruct((NW, N), jnp.float32),
      mesh=mesh,
      compiler_params=pltpu.CompilerParams(needs_layout_passes=False),
      scratch_types=[
          pltpu.VMEM((ECH, 2, 128), jnp.int32),
          pltpu.VMEM((EPW,), jnp.float32),
          pltpu.VMEM((EPW,), jnp.int32),
          pltpu.VMEM((EPW,), jnp.float32),
          pltpu.VMEM((N,), jnp.float32),
          pltpu.SemaphoreType.DMA,
          pltpu.SemaphoreType.DMA,
          pltpu.SemaphoreType.DMA,
          pltpu.SemaphoreType.DMA,
          pltpu.SemaphoreType.DMA,
      ],
  )
  def body(xflat_h, adj_h, edges_h, out_h,
           edges_v, adj_v, idx_v, gath_v, acc_v, sem_in, *sem_w):
    wid = lax.axis_index("s") * NC + lax.axis_index("c")
    cp_e = pltpu.async_copy(edges_h.at[pl.ds(wid * ECH, ECH)], edges_v,
                            sem_in)
    cp_a = pltpu.async_copy(adj_h.at[pl.ds(wid * EPW, EPW)], adj_v, sem_in)

    def zero(i, carry):
      for k in range(4):
        acc_v[pl.ds(i * 4 * L + k * L, L)] = jnp.zeros((L,), jnp.float32)
      return carry
    lax.fori_loop(0, N // (4 * L), zero, 0)

    cp_e.wait()
    cp_a.wait()

    def idx_step(i, carry):
      t = i >> 3
      kofs = (i & 7) * L
      r = edges_v[t, 0, pl.ds(kofs, L)]
      c = edges_v[t, 1, pl.ds(kofs, L)]
      idx_v[pl.ds(i * L, L)] = (
          ((r + ((r >> 3) * 248)) << 7) + (c + ((c >> 7) * 896)))
      return carry

    NWAVE = len(sem_w)
    CPW = NCH // NWAVE
    gathers = []
    for j in range(NCH):
      lax.fori_loop(j * VPC, (j + 1) * VPC, idx_step, 0)
      s = pl.ds(j * GW, GW)
      gathers.append(pltpu.async_copy(
          xflat_h.at[idx_v.at[s]], gath_v.at[s], sem_w[j // CPW]))

    def scat_step(i, carry):
      t = i >> 3
      kofs = (i & 7) * L
      s = pl.ds(i * L, L)
      r = edges_v[t, 0, pl.ds(kofs, L)]
      plsc.addupdate_scatter(acc_v, [r], gath_v[s] * adj_v[s])
      return carry

    WVREGS = CPW * VPC
    for w in range(NWAVE):
      for cp in gathers[w * CPW:(w + 1) * CPW]:
        cp.wait()
      lax.fori_loop(w * WVREGS, (w + 1) * WVREGS, scat_step, 0)

    pltpu.sync_copy(acc_v, out_h.at[wid])

  return body(xflat, adj, edges3)


def _tc_combine_softmax(partials):
  def body(p_ref, araw_ref, alpha_ref):
    a = jnp.sum(p_ref[...], axis=0, keepdims=True)
    araw_ref[...] = a
    m = jnp.max(a)
    e = jnp.exp(a - m)
    alpha_ref[...] = e / jnp.sum(e)

  araw, alpha = pl.pallas_call(
      body,
      out_shape=(jax.ShapeDtypeStruct((1, N), jnp.float32),
                 jax.ShapeDtypeStruct((1, N), jnp.float32)),
  )(partials)
  return araw, alpha


def kernel(data_input, adj_values, edge_index):
  xflat = (data_input.reshape(N // 8, 8, N // 128, 128)
           .swapaxes(1, 2).reshape(N * N))
  edges3 = (edge_index.reshape(2, E // 128, 128).swapaxes(0, 1))
  partials = _sc_segment_partials(xflat, adj_values, edges3)
  araw, alpha = _tc_combine_softmax(partials)
  return alpha.reshape(N), araw.reshape(N)
